# baseline (speedup 1.0000x reference)
import functools

import jax
import jax.numpy as jnp
from jax import lax
from jax.experimental import pallas as pl
from jax.experimental.pallas import tpu as pltpu
from jax.experimental.pallas import tpu_sc as plsc

N_NODES = 10000
N_EDGES = 320000
DIM_IN = 128
DIM_H = 256
DIM_OUT = 128

NUM_SC = 2
NUM_TILES = 16
NW = NUM_SC * NUM_TILES
EDGES_PER_TILE = N_EDGES // NW
CHUNK = 80
NUM_CHUNKS = EDGES_PER_TILE // CHUNK
ROW_STRIPE = 624
ROW_TAIL = N_NODES - ROW_STRIPE * NUM_TILES
LANES = 16


def _sc_scatter_add(feat, eidx, with_count):
    width = feat.shape[1]
    mesh = plsc.VectorSubcoreMesh(core_axis_name="c", subcore_axis_name="s")

    NIDX = 4
    NROW = 3
    out_type = [jax.ShapeDtypeStruct((NUM_SC, N_NODES, width), jnp.float32)]
    scratch = (
        [pltpu.VMEM((2, CHUNK), jnp.int32) for _ in range(NIDX)]
        + [pltpu.VMEM((CHUNK, width), jnp.float32) for _ in range(NROW)]
        + [pltpu.VMEM_SHARED((N_NODES, width), jnp.float32)]
        + [pltpu.SemaphoreType.DMA for _ in range(NIDX + NROW)]
    )
    if with_count:
        out_type.append(jax.ShapeDtypeStruct((NW, N_NODES), jnp.float32))
        scratch.append(pltpu.VMEM((N_NODES,), jnp.float32))

    @functools.partial(
        pl.kernel, out_type=tuple(out_type), mesh=mesh,
        scratch_types=scratch,
        compiler_params=pltpu.CompilerParams(needs_layout_passes=False))
    def k(feat_hbm, eidx_hbm, *rest):
        if with_count:
            (out_hbm, cnt_hbm, *rest) = rest
            cnt_v = rest[-1]
            rest = rest[:-1]
        else:
            (out_hbm, *rest) = rest
        q = rest[:NIDX]
        r = rest[NIDX:NIDX + NROW]
        acc = rest[NIDX + NROW]
        sem_i = rest[NIDX + NROW + 1:NIDX + NROW + 1 + NIDX]
        sem_r = rest[NIDX + NROW + 1 + NIDX:NIDX + NROW + 1 + NIDX + NROW]
        cid = lax.axis_index("c")
        sid = lax.axis_index("s")
        wid = cid * NUM_TILES + sid
        r0 = sid * ROW_STRIPE
        rtail = ROW_STRIPE * NUM_TILES
        ones16 = jnp.full((LANES,), 1.0, jnp.float32)

        ebase = wid * EDGES_PER_TILE

        def fire_idx(i, b):
            off = ebase + i * CHUNK
            pltpu.async_copy(eidx_hbm.at[pl.ds(off, CHUNK)],
                             q[b].at[0], sem_i[b])
            pltpu.async_copy(eidx_hbm.at[pl.ds(N_EDGES + off, CHUNK)],
                             q[b].at[1], sem_i[b])

        def wait_idx(b):
            pltpu.make_async_copy(eidx_hbm.at[pl.ds(0, CHUNK)], q[b].at[0],
                                  sem_i[b]).wait()
            pltpu.make_async_copy(eidx_hbm.at[pl.ds(0, CHUNK)], q[b].at[1],
                                  sem_i[b]).wait()

        def fire_rows(bq, br):
            pltpu.async_copy(feat_hbm.at[q[bq].at[0]], r[br], sem_r[br])

        def wait_rows(br):
            pltpu.make_async_copy(feat_hbm.at[q[0].at[0]], r[br],
                                  sem_r[br]).wait()

        def scatter(bq, br):
            pltpu.sync_copy(r[br], acc.at[q[bq].at[1]], add=True)
            if with_count:
                for j in range(CHUNK // LANES):
                    idx16 = q[bq][1, pl.ds(j * LANES, LANES)]
                    plsc.addupdate_scatter(cnt_v, [idx16], ones16)

        fire_idx(0, 0)
        fire_idx(1, 1)
        fire_idx(2, 2)
        wait_idx(0)
        fire_rows(0, 0)
        wait_idx(1)
        fire_rows(1, 1)

        zb = r[NROW - 1]

        def zero_row(i, _):
            for t in range(width // LANES):
                zb[i, pl.ds(t * LANES, LANES)] = jnp.zeros((LANES,),
                                                           jnp.float32)
            return 0

        lax.fori_loop(0, CHUNK, zero_row, 0)
        for sblk in range(ROW_STRIPE // CHUNK):
            pltpu.sync_copy(zb, acc.at[pl.ds(r0 + sblk * CHUNK, CHUNK)])
        rem = ROW_STRIPE - (ROW_STRIPE // CHUNK) * CHUNK
        pltpu.sync_copy(zb.at[pl.ds(0, rem)],
                        acc.at[pl.ds(r0 + ROW_STRIPE - rem, rem)])

        @pl.when(sid == NUM_TILES - 1)
        def _():
            pltpu.sync_copy(zb.at[pl.ds(0, ROW_TAIL)],
                            acc.at[pl.ds(rtail, ROW_TAIL)])

        if with_count:
            def zero_cnt(i, _):
                cnt_v[pl.ds(i * LANES, LANES)] = jnp.zeros((LANES,),
                                                           jnp.float32)
                return 0
            lax.fori_loop(0, N_NODES // LANES, zero_cnt, 0)

        plsc.subcore_barrier()

        UNROLL = 12

        def block_body(jb, _):
            s0 = jb * UNROLL
            for t in range(UNROLL):
                s = s0 + t
                fire_idx(s + 3, (t + 3) % NIDX)
                wait_idx((t + 2) % NIDX)
                fire_rows((t + 2) % NIDX, (t + 2) % NROW)
                wait_rows(t % NROW)
                scatter(t % NIDX, t % NROW)
            return 0

        lax.fori_loop(0, NUM_CHUNKS // UNROLL, block_body, 0)
        for s in range(UNROLL * (NUM_CHUNKS // UNROLL), NUM_CHUNKS):
            if s + 3 < NUM_CHUNKS:
                fire_idx(s + 3, (s + 3) % NIDX)
            if s + 2 < NUM_CHUNKS:
                wait_idx((s + 2) % NIDX)
                fire_rows((s + 2) % NIDX, (s + 2) % NROW)
            wait_rows(s % NROW)
            scatter(s % NIDX, s % NROW)
        plsc.subcore_barrier()
        pltpu.sync_copy(acc.at[pl.ds(r0, ROW_STRIPE)],
                        out_hbm.at[cid, pl.ds(r0, ROW_STRIPE)])

        @pl.when(sid == NUM_TILES - 1)
        def _():
            pltpu.sync_copy(acc.at[pl.ds(rtail, ROW_TAIL)],
                            out_hbm.at[cid, pl.ds(rtail, ROW_TAIL)])

        if with_count:
            pltpu.sync_copy(cnt_v, cnt_hbm.at[wid])

    return k(feat, eidx)


_ROW_BLK = 1024


def _tc_lin_body(x_ref, w_ref, b_ref, out_ref):
    out_ref[...] = x_ref[...] @ w_ref[...] + b_ref[...]


def _tc_lin(xin, w, b):
    din, dout = w.shape
    return pl.pallas_call(
        _tc_lin_body,
        grid=(pl.cdiv(N_NODES, _ROW_BLK),),
        in_specs=[
            pl.BlockSpec((_ROW_BLK, din), lambda i: (i, 0)),
            pl.BlockSpec((din, dout), lambda i: (0, 0)),
            pl.BlockSpec((dout,), lambda i: (0,)),
        ],
        out_specs=pl.BlockSpec((_ROW_BLK, dout), lambda i: (i, 0)),
        out_shape=jax.ShapeDtypeStruct((N_NODES, dout), jnp.float32),
    )(xin, w, b)


def _invc_col(cnt_ref):
    cnt = jnp.sum(cnt_ref[...], axis=0)
    return 1.0 / jnp.maximum(cnt, 1.0)[:, None]


def _tc1_body(agg_ref, cnt_ref, xr_ref, w1l_ref, w_ref, b_ref, out_ref):
    mean = (agg_ref[0] + agg_ref[1]) * _invc_col(cnt_ref)
    h = jnp.maximum(mean @ w1l_ref[...] + xr_ref[...], 0.0)
    out_ref[...] = h @ w_ref[...] + b_ref[...]


def _tc2_body(agg2_ref, cnt_ref, hr_ref, o_ref, ls_ref):
    o = (agg2_ref[0] + agg2_ref[1]) * _invc_col(cnt_ref) + hr_ref[...]
    o_ref[...] = o
    m = jnp.max(o, axis=1, keepdims=True)
    e = jnp.exp(o - m)
    ls_ref[...] = (o - m) - jnp.log(jnp.sum(e, axis=1, keepdims=True))


def kernel(x, edge_index, W1_l, b1, W1_r, W2_l, b2, W2_r):
    eidx = edge_index.astype(jnp.int32).reshape(-1)

    agg1, cnt_parts = _sc_scatter_add(x, eidx, with_count=True)
    xr = _tc_lin(x, W1_r, b1)

    grid = pl.cdiv(N_NODES, _ROW_BLK)

    def _layer2_proj(w, b):
        return pl.pallas_call(
            _tc1_body,
            grid=(grid,),
            in_specs=[
                pl.BlockSpec((NUM_SC, _ROW_BLK, DIM_IN),
                             lambda i: (0, i, 0)),
                pl.BlockSpec((NW, _ROW_BLK), lambda i: (0, i)),
                pl.BlockSpec((_ROW_BLK, DIM_H), lambda i: (i, 0)),
                pl.BlockSpec((DIM_IN, DIM_H), lambda i: (0, 0)),
                pl.BlockSpec((DIM_H, DIM_OUT), lambda i: (0, 0)),
                pl.BlockSpec((DIM_OUT,), lambda i: (0,)),
            ],
            out_specs=pl.BlockSpec((_ROW_BLK, DIM_OUT), lambda i: (i, 0)),
            out_shape=jax.ShapeDtypeStruct((N_NODES, DIM_OUT), jnp.float32),
        )(agg1, cnt_parts, xr, W1_l, w, b)

    zero_b = jnp.zeros((DIM_OUT,), jnp.float32)
    p = _layer2_proj(W2_l, zero_b)

    agg2, = _sc_scatter_add(p, eidx, with_count=False)
    hr = _layer2_proj(W2_r, b2)

    o, ls = pl.pallas_call(
        _tc2_body,
        grid=(grid,),
        in_specs=[
            pl.BlockSpec((NUM_SC, _ROW_BLK, DIM_OUT), lambda i: (0, i, 0)),
            pl.BlockSpec((NW, _ROW_BLK), lambda i: (0, i)),
            pl.BlockSpec((_ROW_BLK, DIM_OUT), lambda i: (i, 0)),
        ],
        out_specs=[
            pl.BlockSpec((_ROW_BLK, DIM_OUT), lambda i: (i, 0)),
            pl.BlockSpec((_ROW_BLK, DIM_OUT), lambda i: (i, 0)),
        ],
        out_shape=[
            jax.ShapeDtypeStruct((N_NODES, DIM_OUT), jnp.float32),
            jax.ShapeDtypeStruct((N_NODES, DIM_OUT), jnp.float32),
        ],
    )(agg2, cnt_parts, hr)

    return (o, ls)


# speedup vs baseline: 1.0009x; 1.0009x over previous
import functools

import jax
import jax.numpy as jnp
from jax import lax
from jax.experimental import pallas as pl
from jax.experimental.pallas import tpu as pltpu
from jax.experimental.pallas import tpu_sc as plsc

N_NODES = 10000
N_EDGES = 320000
DIM_IN = 128
DIM_H = 256
DIM_OUT = 128

NUM_SC = 2
NUM_TILES = 16
NW = NUM_SC * NUM_TILES
EDGES_PER_TILE = N_EDGES // NW
CHUNK = 80
NUM_CHUNKS = EDGES_PER_TILE // CHUNK
ROW_STRIPE = 624
ROW_TAIL = N_NODES - ROW_STRIPE * NUM_TILES
LANES = 16


def _sc_scatter_add(feat, eidx, with_count):
    width = feat.shape[1]
    mesh = plsc.VectorSubcoreMesh(core_axis_name="c", subcore_axis_name="s")

    NIDX = 4
    NROW = 3
    out_type = [jax.ShapeDtypeStruct((NUM_SC, N_NODES, width), jnp.float32)]
    scratch = (
        [pltpu.VMEM((2, CHUNK), jnp.int32) for _ in range(NIDX)]
        + [pltpu.VMEM((CHUNK, width), jnp.float32) for _ in range(NROW)]
        + [pltpu.VMEM_SHARED((N_NODES, width), jnp.float32)]
        + [pltpu.SemaphoreType.DMA for _ in range(NIDX + NROW)]
    )
    if with_count:
        out_type.append(jax.ShapeDtypeStruct((NW, N_NODES), jnp.float32))
        scratch.append(pltpu.VMEM((N_NODES,), jnp.float32))

    @functools.partial(
        pl.kernel, out_type=tuple(out_type), mesh=mesh,
        scratch_types=scratch,
        compiler_params=pltpu.CompilerParams(needs_layout_passes=False))
    def k(feat_hbm, eidx_hbm, *rest):
        if with_count:
            (out_hbm, cnt_hbm, *rest) = rest
            cnt_v = rest[-1]
            rest = rest[:-1]
        else:
            (out_hbm, *rest) = rest
        q = rest[:NIDX]
        r = rest[NIDX:NIDX + NROW]
        acc = rest[NIDX + NROW]
        sem_i = rest[NIDX + NROW + 1:NIDX + NROW + 1 + NIDX]
        sem_r = rest[NIDX + NROW + 1 + NIDX:NIDX + NROW + 1 + NIDX + NROW]
        cid = lax.axis_index("c")
        sid = lax.axis_index("s")
        wid = cid * NUM_TILES + sid
        r0 = sid * ROW_STRIPE
        rtail = ROW_STRIPE * NUM_TILES
        ones16 = jnp.full((LANES,), 1.0, jnp.float32)

        ebase = wid * EDGES_PER_TILE

        def fire_idx(i, b):
            off = ebase + i * CHUNK
            pltpu.async_copy(eidx_hbm.at[pl.ds(off, CHUNK)],
                             q[b].at[0], sem_i[b])
            pltpu.async_copy(eidx_hbm.at[pl.ds(N_EDGES + off, CHUNK)],
                             q[b].at[1], sem_i[b])

        def wait_idx(b):
            pltpu.make_async_copy(eidx_hbm.at[pl.ds(0, CHUNK)], q[b].at[0],
                                  sem_i[b]).wait()
            pltpu.make_async_copy(eidx_hbm.at[pl.ds(0, CHUNK)], q[b].at[1],
                                  sem_i[b]).wait()

        def fire_rows(bq, br):
            pltpu.async_copy(feat_hbm.at[q[bq].at[0]], r[br], sem_r[br])

        def wait_rows(br):
            pltpu.make_async_copy(feat_hbm.at[q[0].at[0]], r[br],
                                  sem_r[br]).wait()

        def scatter(bq, br):
            pltpu.sync_copy(r[br], acc.at[q[bq].at[1]], add=True)
            if with_count:
                for j in range(CHUNK // LANES):
                    idx16 = q[bq][1, pl.ds(j * LANES, LANES)]
                    plsc.addupdate_scatter(cnt_v, [idx16], ones16)

        fire_idx(0, 0)
        fire_idx(1, 1)
        fire_idx(2, 2)
        wait_idx(0)
        fire_rows(0, 0)
        wait_idx(1)
        fire_rows(1, 1)

        zb = r[NROW - 1]

        def zero_row(i, _):
            for t in range(width // LANES):
                zb[i, pl.ds(t * LANES, LANES)] = jnp.zeros((LANES,),
                                                           jnp.float32)
            return 0

        lax.fori_loop(0, CHUNK, zero_row, 0)
        for sblk in range(ROW_STRIPE // CHUNK):
            pltpu.sync_copy(zb, acc.at[pl.ds(r0 + sblk * CHUNK, CHUNK)])
        rem = ROW_STRIPE - (ROW_STRIPE // CHUNK) * CHUNK
        pltpu.sync_copy(zb.at[pl.ds(0, rem)],
                        acc.at[pl.ds(r0 + ROW_STRIPE - rem, rem)])

        @pl.when(sid == NUM_TILES - 1)
        def _():
            pltpu.sync_copy(zb.at[pl.ds(0, ROW_TAIL)],
                            acc.at[pl.ds(rtail, ROW_TAIL)])

        if with_count:
            def zero_cnt(i, _):
                cnt_v[pl.ds(i * LANES, LANES)] = jnp.zeros((LANES,),
                                                           jnp.float32)
                return 0
            lax.fori_loop(0, N_NODES // LANES, zero_cnt, 0)

        plsc.subcore_barrier()

        UNROLL = 12

        def block_body(jb, _):
            s0 = jb * UNROLL
            for t in range(UNROLL):
                s = s0 + t
                fire_idx(s + 3, (t + 3) % NIDX)
                wait_idx((t + 2) % NIDX)
                fire_rows((t + 2) % NIDX, (t + 2) % NROW)
                wait_rows(t % NROW)
                scatter(t % NIDX, t % NROW)
            return 0

        lax.fori_loop(0, NUM_CHUNKS // UNROLL, block_body, 0)
        for s in range(UNROLL * (NUM_CHUNKS // UNROLL), NUM_CHUNKS):
            if s + 3 < NUM_CHUNKS:
                fire_idx(s + 3, (s + 3) % NIDX)
            if s + 2 < NUM_CHUNKS:
                wait_idx((s + 2) % NIDX)
                fire_rows((s + 2) % NIDX, (s + 2) % NROW)
            wait_rows(s % NROW)
            scatter(s % NIDX, s % NROW)
        plsc.subcore_barrier()
        pltpu.sync_copy(acc.at[pl.ds(r0, ROW_STRIPE)],
                        out_hbm.at[cid, pl.ds(r0, ROW_STRIPE)])

        @pl.when(sid == NUM_TILES - 1)
        def _():
            pltpu.sync_copy(acc.at[pl.ds(rtail, ROW_TAIL)],
                            out_hbm.at[cid, pl.ds(rtail, ROW_TAIL)])

        if with_count:
            pltpu.sync_copy(cnt_v, cnt_hbm.at[wid])

    return k(feat, eidx)


_ROW_BLK = 1024


def _tc_lin_body(x_ref, w_ref, b_ref, out_ref):
    out_ref[...] = x_ref[...] @ w_ref[...] + b_ref[...]


def _tc_lin(xin, w, b):
    din, dout = w.shape
    return pl.pallas_call(
        _tc_lin_body,
        grid=(pl.cdiv(N_NODES, _ROW_BLK),),
        in_specs=[
            pl.BlockSpec((_ROW_BLK, din), lambda i: (i, 0)),
            pl.BlockSpec((din, dout), lambda i: (0, 0)),
            pl.BlockSpec((dout,), lambda i: (0,)),
        ],
        out_specs=pl.BlockSpec((_ROW_BLK, dout), lambda i: (i, 0)),
        out_shape=jax.ShapeDtypeStruct((N_NODES, dout), jnp.float32),
    )(xin, w, b)


def _invc_col(cnt_ref):
    cnt = jnp.sum(cnt_ref[...], axis=0)
    return 1.0 / jnp.maximum(cnt, 1.0)[:, None]


def _tc1_body(agg_ref, cnt_ref, xr_ref, w1l_ref, w_ref, out_ref):
    mean = (agg_ref[0] + agg_ref[1]) * _invc_col(cnt_ref)
    h = jnp.maximum(mean @ w1l_ref[...] + xr_ref[...], 0.0)
    out_ref[...] = h @ w_ref[...]


def _tc2_body(agg2_ref, cnt_ref, hr_ref, b2_ref, o_ref, ls_ref):
    o = ((agg2_ref[0] + agg2_ref[1]) * _invc_col(cnt_ref)
         + hr_ref[...] + b2_ref[...])
    o_ref[...] = o
    m = jnp.max(o, axis=1, keepdims=True)
    e = jnp.exp(o - m)
    ls_ref[...] = (o - m) - jnp.log(jnp.sum(e, axis=1, keepdims=True))


def kernel(x, edge_index, W1_l, b1, W1_r, W2_l, b2, W2_r):
    eidx = edge_index.astype(jnp.int32).reshape(-1)

    agg1, cnt_parts = _sc_scatter_add(x, eidx, with_count=True)
    xr = _tc_lin(x, W1_r, b1)

    grid = pl.cdiv(N_NODES, _ROW_BLK)

    def _layer2_proj(w):
        return pl.pallas_call(
            _tc1_body,
            grid=(grid,),
            in_specs=[
                pl.BlockSpec((NUM_SC, _ROW_BLK, DIM_IN),
                             lambda i: (0, i, 0)),
                pl.BlockSpec((NW, _ROW_BLK), lambda i: (0, i)),
                pl.BlockSpec((_ROW_BLK, DIM_H), lambda i: (i, 0)),
                pl.BlockSpec((DIM_IN, DIM_H), lambda i: (0, 0)),
                pl.BlockSpec((DIM_H, DIM_OUT), lambda i: (0, 0)),
            ],
            out_specs=pl.BlockSpec((_ROW_BLK, DIM_OUT), lambda i: (i, 0)),
            out_shape=jax.ShapeDtypeStruct((N_NODES, DIM_OUT), jnp.float32),
        )(agg1, cnt_parts, xr, W1_l, w)

    p = _layer2_proj(W2_l)

    agg2, = _sc_scatter_add(p, eidx, with_count=False)
    hr = _layer2_proj(W2_r)

    o, ls = pl.pallas_call(
        _tc2_body,
        grid=(grid,),
        in_specs=[
            pl.BlockSpec((NUM_SC, _ROW_BLK, DIM_OUT), lambda i: (0, i, 0)),
            pl.BlockSpec((NW, _ROW_BLK), lambda i: (0, i)),
            pl.BlockSpec((_ROW_BLK, DIM_OUT), lambda i: (i, 0)),
            pl.BlockSpec((DIM_OUT,), lambda i: (0,)),
        ],
        out_specs=[
            pl.BlockSpec((_ROW_BLK, DIM_OUT), lambda i: (i, 0)),
            pl.BlockSpec((_ROW_BLK, DIM_OUT), lambda i: (i, 0)),
        ],
        out_shape=[
            jax.ShapeDtypeStruct((N_NODES, DIM_OUT), jnp.float32),
            jax.ShapeDtypeStruct((N_NODES, DIM_OUT), jnp.float32),
        ],
    )(agg2, cnt_parts, hr, b2)

    return (o, ls)
